# baseline (speedup 1.0000x reference)
import jax
import jax.numpy as jnp
from jax import lax
from jax.experimental import pallas as pl
from jax.experimental.pallas import tpu as pltpu
from jax.experimental.pallas import tpu_sc as plsc

H = 64
W = 64
NW = 32
CHUNK = 8
LANES = 16


def _sc_kernel(fov_hbm, act_hbm, pos_hbm, tab_hbm, val_hbm,
               out_hbm, pos_out_hbm, mask_out_hbm,
               buf0, buf1, abuf, pbuf, pobuf, mobuf, tabbuf, valbuf,
               si0, si1, so0, so1, sm):
    B = fov_hbm.shape[0]
    rpw = B // NW
    nch = rpw // CHUNK
    wid = lax.axis_index("s") * 2 + lax.axis_index("c")
    base = wid * rpw

    pltpu.make_async_copy(act_hbm.at[pl.ds(base, rpw)],
                          abuf.at[pl.ds(0, rpw)], sm).start()
    pltpu.make_async_copy(act_hbm.at[pl.ds(base, rpw)],
                          abuf.at[pl.ds(0, rpw)], sm).wait()
    pltpu.make_async_copy(pos_hbm.at[pl.ds(2 * base, 2 * rpw)],
                          pbuf.at[pl.ds(0, 2 * rpw)], sm).start()
    pltpu.make_async_copy(pos_hbm.at[pl.ds(2 * base, 2 * rpw)],
                          pbuf.at[pl.ds(0, 2 * rpw)], sm).wait()
    pltpu.make_async_copy(tab_hbm, tabbuf, sm).start()
    pltpu.make_async_copy(tab_hbm, tabbuf, sm).wait()
    pltpu.make_async_copy(val_hbm, valbuf, sm).start()
    pltpu.make_async_copy(val_hbm, valbuf, sm).wait()

    k16 = lax.iota(jnp.int32, LANES)
    lane_ok = k16 < CHUNK
    krow = jnp.where(lane_ok, k16, 0)
    zeros = jnp.zeros((LANES,), jnp.int32)
    val_vec = valbuf[...]

    def start_in(g, buf, sem):
        pltpu.make_async_copy(
            fov_hbm.at[pl.ds(base + g * CHUNK, CHUNK)], buf, sem).start()

    def wait_in(g, buf, sem):
        pltpu.make_async_copy(
            fov_hbm.at[pl.ds(base + g * CHUNK, CHUNK)], buf, sem).wait()

    def start_out(g, buf, sem):
        pltpu.make_async_copy(
            buf, out_hbm.at[pl.ds(base + g * CHUNK, CHUNK)], sem).start()

    def wait_out(g, buf, sem):
        pltpu.make_async_copy(
            buf, out_hbm.at[pl.ds(base + g * CHUNK, CHUNK)], sem).wait()

    def process(g, buf):
        rl = g * CHUNK
        aidx = plsc.load_gather(abuf, [jnp.where(lane_ok, rl + k16, 0)])
        aidx = jnp.clip(aidx, 0, 8)
        ys = plsc.load_gather(pbuf, [jnp.where(lane_ok, 2 * (rl + k16), 0)])
        xs = plsc.load_gather(pbuf, [jnp.where(lane_ok, 2 * (rl + k16) + 1, 0)])
        dy = plsc.load_gather(tabbuf, [2 * aidx])
        dx = plsc.load_gather(tabbuf, [2 * aidx + 1])
        ny = jnp.clip(ys + dy, 0, H - 1)
        nx = jnp.clip(xs + dx, 0, W - 1)
        f1 = jnp.where(lane_ok, ny * W + nx, 0)
        cell = plsc.load_gather(buf, [krow, f1], mask=lane_ok)
        blocked = cell == 1.0
        target = cell == 2.0
        dy2 = jnp.where(blocked, 0, dy)
        dx2 = jnp.where(blocked, 0, dx)
        y2 = ys + dy2
        x2 = xs + dx2
        f2 = jnp.where(
            lane_ok, jnp.clip(y2, 0, H - 1) * W + jnp.clip(x2, 0, W - 1), 0)
        plsc.store_scatter(buf, [krow, f2], val_vec, mask=lane_ok)
        plsc.store_scatter(pobuf, [jnp.where(lane_ok, 2 * (rl + k16), 0)],
                           y2, mask=lane_ok)
        plsc.store_scatter(pobuf, [jnp.where(lane_ok, 2 * (rl + k16) + 1, 0)],
                           x2, mask=lane_ok)
        plsc.store_scatter(mobuf, [jnp.where(lane_ok, rl + k16, 0)],
                           jnp.where(target, 1, 0), mask=lane_ok)

    start_in(0, buf0, si0)
    start_in(1, buf1, si1)

    def body(i, carry):
        g = i * 2
        wait_in(g, buf0, si0)
        process(g, buf0)
        start_out(g, buf0, so0)
        wait_in(g + 1, buf1, si1)
        process(g + 1, buf1)
        start_out(g + 1, buf1, so1)

        @pl.when(i < nch // 2 - 1)
        def _():
            wait_out(g, buf0, so0)
            start_in(g + 2, buf0, si0)
            wait_out(g + 1, buf1, so1)
            start_in(g + 3, buf1, si1)
        return carry

    lax.fori_loop(0, nch // 2, body, 0)

    pltpu.make_async_copy(pobuf.at[pl.ds(0, 2 * rpw)],
                          pos_out_hbm.at[pl.ds(2 * base, 2 * rpw)], sm).start()
    pltpu.make_async_copy(pobuf.at[pl.ds(0, 2 * rpw)],
                          pos_out_hbm.at[pl.ds(2 * base, 2 * rpw)], sm).wait()
    pltpu.make_async_copy(mobuf.at[pl.ds(0, rpw)],
                          mask_out_hbm.at[pl.ds(base, rpw)], sm).start()
    pltpu.make_async_copy(mobuf.at[pl.ds(0, rpw)],
                          mask_out_hbm.at[pl.ds(base, rpw)], sm).wait()
    wait_out(nch - 2, buf0, so0)
    wait_out(nch - 1, buf1, so1)


def _sc_run(fov2d, act_flat, pos_flat, tab_pad, val_arr):
    B = fov2d.shape[0]
    rpw = B // NW
    mesh = plsc.VectorSubcoreMesh(core_axis_name="c", subcore_axis_name="s")
    f = pl.kernel(
        _sc_kernel,
        out_type=[
            jax.ShapeDtypeStruct((B, H * W), jnp.float32),
            jax.ShapeDtypeStruct((2 * B,), jnp.int32),
            jax.ShapeDtypeStruct((B,), jnp.int32),
        ],
        mesh=mesh,
        compiler_params=pltpu.CompilerParams(needs_layout_passes=False),
        scratch_types=[
            pltpu.VMEM((CHUNK, H * W), jnp.float32),
            pltpu.VMEM((CHUNK, H * W), jnp.float32),
            pltpu.VMEM((rpw + LANES,), jnp.int32),
            pltpu.VMEM((2 * rpw + 2 * LANES,), jnp.int32),
            pltpu.VMEM((2 * rpw + 2 * LANES,), jnp.int32),
            pltpu.VMEM((rpw + LANES,), jnp.int32),
            pltpu.VMEM((32,), jnp.int32),
            pltpu.VMEM((LANES,), jnp.float32),
            pltpu.SemaphoreType.DMA,
            pltpu.SemaphoreType.DMA,
            pltpu.SemaphoreType.DMA,
            pltpu.SemaphoreType.DMA,
            pltpu.SemaphoreType.DMA,
        ],
    )
    return f(fov2d, act_flat, pos_flat, tab_pad, val_arr)


def kernel(fov, batch_logit_prob, batch_top_k_prob, batch_action_idx,
           possible_actions, batch_agent_current_pos, step):
    B = fov.shape[0]
    val_arr = jnp.full((LANES,), 3.0 + jnp.asarray(step, jnp.float32),
                       jnp.float32)
    tab_pad = jnp.zeros((32,), jnp.int32).at[:18].set(
        possible_actions.reshape(18))
    new_fov, pos_out, tmask = _sc_run(
        fov.reshape(B, H * W),
        batch_action_idx.reshape(B),
        batch_agent_current_pos.reshape(2 * B),
        tab_pad,
        val_arr)
    return (new_fov.reshape(B, H, W), pos_out.reshape(B, 2),
            tmask.astype(bool),
            batch_action_idx, batch_logit_prob, batch_top_k_prob)


# speedup vs baseline: 1.0014x; 1.0014x over previous
import jax
import jax.numpy as jnp
from jax import lax
from jax.experimental import pallas as pl
from jax.experimental.pallas import tpu as pltpu
from jax.experimental.pallas import tpu_sc as plsc

H = 64
W = 64
NW = 32
CHUNK = 8
LANES = 16


def _sc_kernel(fov_hbm, act_hbm, pos_hbm, tab_hbm, val_hbm,
               out_hbm, pos_out_hbm, mask_out_hbm,
               buf0, buf1, abuf, pbuf, pobuf, mobuf, tabbuf, valbuf,
               ysbuf, xsbuf, dybuf, dxbuf, f1buf,
               si0, si1, so0, so1, sm):
    B = fov_hbm.shape[0]
    rpw = B // NW
    nch = rpw // CHUNK
    wid = lax.axis_index("s") * 2 + lax.axis_index("c")
    base = wid * rpw

    def start_in(g, buf, sem):
        pltpu.make_async_copy(
            fov_hbm.at[pl.ds(base + g * CHUNK, CHUNK)], buf, sem).start()

    def wait_in(g, buf, sem):
        pltpu.make_async_copy(
            fov_hbm.at[pl.ds(base + g * CHUNK, CHUNK)], buf, sem).wait()

    def start_out(g, buf, sem):
        pltpu.make_async_copy(
            buf, out_hbm.at[pl.ds(base + g * CHUNK, CHUNK)], sem).start()

    def wait_out(g, buf, sem):
        pltpu.make_async_copy(
            buf, out_hbm.at[pl.ds(base + g * CHUNK, CHUNK)], sem).wait()

    start_in(0, buf0, si0)
    start_in(1, buf1, si1)

    pltpu.make_async_copy(act_hbm.at[pl.ds(base, rpw)],
                          abuf.at[pl.ds(0, rpw)], sm).start()
    pltpu.make_async_copy(act_hbm.at[pl.ds(base, rpw)],
                          abuf.at[pl.ds(0, rpw)], sm).wait()
    pltpu.make_async_copy(pos_hbm.at[pl.ds(2 * base, 2 * rpw)],
                          pbuf.at[pl.ds(0, 2 * rpw)], sm).start()
    pltpu.make_async_copy(pos_hbm.at[pl.ds(2 * base, 2 * rpw)],
                          pbuf.at[pl.ds(0, 2 * rpw)], sm).wait()
    pltpu.make_async_copy(tab_hbm, tabbuf, sm).start()
    pltpu.make_async_copy(tab_hbm, tabbuf, sm).wait()
    pltpu.make_async_copy(val_hbm, valbuf, sm).start()
    pltpu.make_async_copy(val_hbm, valbuf, sm).wait()

    k16 = lax.iota(jnp.int32, LANES)
    lane_ok = k16 < CHUNK
    krow = jnp.where(lane_ok, k16, 0)
    val_vec = valbuf[...]

    def pre(j, carry):
        rows = j * LANES + k16
        aidx = jnp.clip(abuf[pl.ds(j * LANES, LANES)], 0, 8)
        ys = plsc.load_gather(pbuf, [2 * rows])
        xs = plsc.load_gather(pbuf, [2 * rows + 1])
        dy = plsc.load_gather(tabbuf, [2 * aidx])
        dx = plsc.load_gather(tabbuf, [2 * aidx + 1])
        ny = jnp.clip(ys + dy, 0, H - 1)
        nx = jnp.clip(xs + dx, 0, W - 1)
        ysbuf[pl.ds(j * LANES, LANES)] = ys
        xsbuf[pl.ds(j * LANES, LANES)] = xs
        dybuf[pl.ds(j * LANES, LANES)] = dy
        dxbuf[pl.ds(j * LANES, LANES)] = dx
        f1buf[pl.ds(j * LANES, LANES)] = ny * W + nx
        return carry

    lax.fori_loop(0, rpw // LANES, pre, 0)

    def process(g, buf):
        rl = g * CHUNK
        f1 = jnp.where(lane_ok, f1buf[pl.ds(rl, LANES)], 0)
        cell = plsc.load_gather(buf, [krow, f1], mask=lane_ok)
        blocked = cell == 1.0
        target = cell == 2.0
        dy2 = jnp.where(blocked, 0, dybuf[pl.ds(rl, LANES)])
        dx2 = jnp.where(blocked, 0, dxbuf[pl.ds(rl, LANES)])
        y2 = ysbuf[pl.ds(rl, LANES)] + dy2
        x2 = xsbuf[pl.ds(rl, LANES)] + dx2
        f2 = jnp.where(
            lane_ok, jnp.clip(y2, 0, H - 1) * W + jnp.clip(x2, 0, W - 1), 0)
        plsc.store_scatter(buf, [krow, f2], val_vec, mask=lane_ok)
        rg = rl + k16
        plsc.store_scatter(pobuf, [jnp.where(lane_ok, 2 * rg, 0)],
                           y2, mask=lane_ok)
        plsc.store_scatter(pobuf, [jnp.where(lane_ok, 2 * rg + 1, 0)],
                           x2, mask=lane_ok)
        plsc.store_scatter(mobuf, [jnp.where(lane_ok, rg, 0)],
                           jnp.where(target, 1, 0), mask=lane_ok)

    def body(i, carry):
        g = i * 2
        wait_in(g, buf0, si0)
        process(g, buf0)
        start_out(g, buf0, so0)
        wait_in(g + 1, buf1, si1)
        process(g + 1, buf1)
        start_out(g + 1, buf1, so1)

        @pl.when(i < nch // 2 - 1)
        def _():
            wait_out(g, buf0, so0)
            start_in(g + 2, buf0, si0)
            wait_out(g + 1, buf1, so1)
            start_in(g + 3, buf1, si1)
        return carry

    lax.fori_loop(0, nch // 2, body, 0)

    pltpu.make_async_copy(pobuf.at[pl.ds(0, 2 * rpw)],
                          pos_out_hbm.at[pl.ds(2 * base, 2 * rpw)], sm).start()
    pltpu.make_async_copy(pobuf.at[pl.ds(0, 2 * rpw)],
                          pos_out_hbm.at[pl.ds(2 * base, 2 * rpw)], sm).wait()
    pltpu.make_async_copy(mobuf.at[pl.ds(0, rpw)],
                          mask_out_hbm.at[pl.ds(base, rpw)], sm).start()
    pltpu.make_async_copy(mobuf.at[pl.ds(0, rpw)],
                          mask_out_hbm.at[pl.ds(base, rpw)], sm).wait()
    wait_out(nch - 2, buf0, so0)
    wait_out(nch - 1, buf1, so1)


def _sc_run(fov2d, act_flat, pos_flat, tab_pad, val_arr):
    B = fov2d.shape[0]
    rpw = B // NW
    mesh = plsc.VectorSubcoreMesh(core_axis_name="c", subcore_axis_name="s")
    meta_i32 = pltpu.VMEM((rpw + LANES,), jnp.int32)
    f = pl.kernel(
        _sc_kernel,
        out_type=[
            jax.ShapeDtypeStruct((B, H * W), jnp.float32),
            jax.ShapeDtypeStruct((2 * B,), jnp.int32),
            jax.ShapeDtypeStruct((B,), jnp.int32),
        ],
        mesh=mesh,
        compiler_params=pltpu.CompilerParams(needs_layout_passes=False),
        scratch_types=[
            pltpu.VMEM((CHUNK, H * W), jnp.float32),
            pltpu.VMEM((CHUNK, H * W), jnp.float32),
            meta_i32,
            pltpu.VMEM((2 * rpw + 2 * LANES,), jnp.int32),
            pltpu.VMEM((2 * rpw + 2 * LANES,), jnp.int32),
            meta_i32,
            pltpu.VMEM((32,), jnp.int32),
            pltpu.VMEM((LANES,), jnp.float32),
            meta_i32,
            meta_i32,
            meta_i32,
            meta_i32,
            meta_i32,
            pltpu.SemaphoreType.DMA,
            pltpu.SemaphoreType.DMA,
            pltpu.SemaphoreType.DMA,
            pltpu.SemaphoreType.DMA,
            pltpu.SemaphoreType.DMA,
        ],
    )
    return f(fov2d, act_flat, pos_flat, tab_pad, val_arr)


def kernel(fov, batch_logit_prob, batch_top_k_prob, batch_action_idx,
           possible_actions, batch_agent_current_pos, step):
    B = fov.shape[0]
    val_arr = jnp.full((LANES,), 3.0 + jnp.asarray(step, jnp.float32),
                       jnp.float32)
    tab_pad = jnp.zeros((32,), jnp.int32).at[:18].set(
        possible_actions.reshape(18))
    new_fov, pos_out, tmask = _sc_run(
        fov.reshape(B, H * W),
        batch_action_idx.reshape(B),
        batch_agent_current_pos.reshape(2 * B),
        tab_pad,
        val_arr)
    return (new_fov.reshape(B, H, W), pos_out.reshape(B, 2),
            tmask.astype(bool),
            batch_action_idx, batch_logit_prob, batch_top_k_prob)


# speedup vs baseline: 1.0051x; 1.0036x over previous
import jax
import jax.numpy as jnp
from jax import lax
from jax.experimental import pallas as pl
from jax.experimental.pallas import tpu as pltpu
from jax.experimental.pallas import tpu_sc as plsc

H = 64
W = 64
NW = 32
CHUNK = 8
LANES = 16


def _sc_kernel(fov_hbm, act_hbm, pos_hbm, tab_hbm, val_hbm,
               out_hbm, pos_out_hbm, mask_out_hbm,
               buf0, buf1, buf2, abuf, pbuf, pobuf, mobuf, tabbuf, valbuf,
               ysbuf, xsbuf, dybuf, dxbuf, f1buf,
               si0, si1, si2, so0, so1, so2, sm):
    B = fov_hbm.shape[0]
    rpw = B // NW
    nch = rpw // CHUNK
    wid = lax.axis_index("s") * 2 + lax.axis_index("c")
    base = wid * rpw

    def start_in(g, buf, sem):
        pltpu.make_async_copy(
            fov_hbm.at[pl.ds(base + g * CHUNK, CHUNK)], buf, sem).start()

    def wait_in(g, buf, sem):
        pltpu.make_async_copy(
            fov_hbm.at[pl.ds(base + g * CHUNK, CHUNK)], buf, sem).wait()

    def start_out(g, buf, sem):
        pltpu.make_async_copy(
            buf, out_hbm.at[pl.ds(base + g * CHUNK, CHUNK)], sem).start()

    def wait_out(g, buf, sem):
        pltpu.make_async_copy(
            buf, out_hbm.at[pl.ds(base + g * CHUNK, CHUNK)], sem).wait()

    start_in(0, buf0, si0)
    start_in(1, buf1, si1)
    start_in(2, buf2, si2)

    pltpu.make_async_copy(act_hbm.at[pl.ds(base, rpw)],
                          abuf.at[pl.ds(0, rpw)], sm).start()
    pltpu.make_async_copy(act_hbm.at[pl.ds(base, rpw)],
                          abuf.at[pl.ds(0, rpw)], sm).wait()
    pltpu.make_async_copy(pos_hbm.at[pl.ds(2 * base, 2 * rpw)],
                          pbuf.at[pl.ds(0, 2 * rpw)], sm).start()
    pltpu.make_async_copy(pos_hbm.at[pl.ds(2 * base, 2 * rpw)],
                          pbuf.at[pl.ds(0, 2 * rpw)], sm).wait()
    pltpu.make_async_copy(tab_hbm, tabbuf, sm).start()
    pltpu.make_async_copy(tab_hbm, tabbuf, sm).wait()
    pltpu.make_async_copy(val_hbm, valbuf, sm).start()
    pltpu.make_async_copy(val_hbm, valbuf, sm).wait()

    k16 = lax.iota(jnp.int32, LANES)
    lane_ok = k16 < CHUNK
    krow = jnp.where(lane_ok, k16, 0)
    val_vec = valbuf[...]

    def pre(j, carry):
        rows = j * LANES + k16
        aidx = jnp.clip(abuf[pl.ds(j * LANES, LANES)], 0, 8)
        ys = plsc.load_gather(pbuf, [2 * rows])
        xs = plsc.load_gather(pbuf, [2 * rows + 1])
        dy = plsc.load_gather(tabbuf, [2 * aidx])
        dx = plsc.load_gather(tabbuf, [2 * aidx + 1])
        ny = jnp.clip(ys + dy, 0, H - 1)
        nx = jnp.clip(xs + dx, 0, W - 1)
        ysbuf[pl.ds(j * LANES, LANES)] = ys
        xsbuf[pl.ds(j * LANES, LANES)] = xs
        dybuf[pl.ds(j * LANES, LANES)] = dy
        dxbuf[pl.ds(j * LANES, LANES)] = dx
        f1buf[pl.ds(j * LANES, LANES)] = ny * W + nx
        return carry

    lax.fori_loop(0, rpw // LANES, pre, 0)

    def process(g, buf):
        rl = g * CHUNK
        f1 = jnp.where(lane_ok, f1buf[pl.ds(rl, LANES)], 0)
        cell = plsc.load_gather(buf, [krow, f1], mask=lane_ok)
        blocked = cell == 1.0
        target = cell == 2.0
        dy2 = jnp.where(blocked, 0, dybuf[pl.ds(rl, LANES)])
        dx2 = jnp.where(blocked, 0, dxbuf[pl.ds(rl, LANES)])
        y2 = ysbuf[pl.ds(rl, LANES)] + dy2
        x2 = xsbuf[pl.ds(rl, LANES)] + dx2
        f2 = jnp.where(
            lane_ok, jnp.clip(y2, 0, H - 1) * W + jnp.clip(x2, 0, W - 1), 0)
        plsc.store_scatter(buf, [krow, f2], val_vec, mask=lane_ok)
        rg = rl + k16
        plsc.store_scatter(pobuf, [jnp.where(lane_ok, 2 * rg, 0)],
                           y2, mask=lane_ok)
        plsc.store_scatter(pobuf, [jnp.where(lane_ok, 2 * rg + 1, 0)],
                           x2, mask=lane_ok)
        plsc.store_scatter(mobuf, [jnp.where(lane_ok, rg, 0)],
                           jnp.where(target, 1, 0), mask=lane_ok)

    slots = ((buf0, si0, so0), (buf1, si1, so1), (buf2, si2, so2))

    def body(i, carry):
        for b, (buf, si, so) in enumerate(slots):
            g = i * 3 + b
            wait_in(g, buf, si)
            process(g, buf)
            start_out(g, buf, so)

            @pl.when(g + 3 < nch)
            def _():
                wait_out(g, buf, so)
                start_in(g + 3, buf, si)
        return carry

    lax.fori_loop(0, nch // 3, body, 0)
    for g in range(nch - nch % 3, nch):
        buf, si, so = slots[g % 3]
        wait_in(g, buf, si)
        process(g, buf)
        start_out(g, buf, so)

    pltpu.make_async_copy(pobuf.at[pl.ds(0, 2 * rpw)],
                          pos_out_hbm.at[pl.ds(2 * base, 2 * rpw)], sm).start()
    pltpu.make_async_copy(pobuf.at[pl.ds(0, 2 * rpw)],
                          pos_out_hbm.at[pl.ds(2 * base, 2 * rpw)], sm).wait()
    pltpu.make_async_copy(mobuf.at[pl.ds(0, rpw)],
                          mask_out_hbm.at[pl.ds(base, rpw)], sm).start()
    pltpu.make_async_copy(mobuf.at[pl.ds(0, rpw)],
                          mask_out_hbm.at[pl.ds(base, rpw)], sm).wait()
    for g in range(nch - 3, nch):
        buf, si, so = ((buf0, si0, so0), (buf1, si1, so1),
                       (buf2, si2, so2))[g % 3]
        wait_out(g, buf, so)


def _sc_run(fov2d, act_flat, pos_flat, tab_pad, val_arr):
    B = fov2d.shape[0]
    rpw = B // NW
    mesh = plsc.VectorSubcoreMesh(core_axis_name="c", subcore_axis_name="s")
    meta_i32 = pltpu.VMEM((rpw + LANES,), jnp.int32)
    f = pl.kernel(
        _sc_kernel,
        out_type=[
            jax.ShapeDtypeStruct((B, H * W), jnp.float32),
            jax.ShapeDtypeStruct((2 * B,), jnp.int32),
            jax.ShapeDtypeStruct((B,), jnp.int32),
        ],
        mesh=mesh,
        compiler_params=pltpu.CompilerParams(needs_layout_passes=False),
        scratch_types=[
            pltpu.VMEM((CHUNK, H * W), jnp.float32),
            pltpu.VMEM((CHUNK, H * W), jnp.float32),
            pltpu.VMEM((CHUNK, H * W), jnp.float32),
            meta_i32,
            pltpu.VMEM((2 * rpw + 2 * LANES,), jnp.int32),
            pltpu.VMEM((2 * rpw + 2 * LANES,), jnp.int32),
            meta_i32,
            pltpu.VMEM((32,), jnp.int32),
            pltpu.VMEM((LANES,), jnp.float32),
            meta_i32,
            meta_i32,
            meta_i32,
            meta_i32,
            meta_i32,
            pltpu.SemaphoreType.DMA,
            pltpu.SemaphoreType.DMA,
            pltpu.SemaphoreType.DMA,
            pltpu.SemaphoreType.DMA,
            pltpu.SemaphoreType.DMA,
            pltpu.SemaphoreType.DMA,
            pltpu.SemaphoreType.DMA,
        ],
    )
    return f(fov2d, act_flat, pos_flat, tab_pad, val_arr)


def kernel(fov, batch_logit_prob, batch_top_k_prob, batch_action_idx,
           possible_actions, batch_agent_current_pos, step):
    B = fov.shape[0]
    val_arr = jnp.full((LANES,), 3.0 + jnp.asarray(step, jnp.float32),
                       jnp.float32)
    tab_pad = jnp.zeros((32,), jnp.int32).at[:18].set(
        possible_actions.reshape(18))
    new_fov, pos_out, tmask = _sc_run(
        fov.reshape(B, H * W),
        batch_action_idx.reshape(B),
        batch_agent_current_pos.reshape(2 * B),
        tab_pad,
        val_arr)
    return (new_fov.reshape(B, H, W), pos_out.reshape(B, 2),
            tmask.astype(bool),
            batch_action_idx, batch_logit_prob, batch_top_k_prob)
